# baseline (speedup 1.0000x reference)
import functools

import jax
import jax.numpy as jnp
from jax import lax
from jax.experimental import pallas as pl
from jax.experimental.pallas import tpu as pltpu
from jax.experimental.pallas import tpu_sc as plsc

NUM_TYPES = 64
D = 128
N = 100000
LANES = 16

NC, NS = 2, 16
NW = NC * NS
C = 112
PW = 3136
NCHUNK = PW // C


def _body(idx_hbm, feat_hbm, table_hbm, out_hbm,
          idx_v, feat_v, gath_v, sem):
    wid = lax.axis_index("s") * NC + lax.axis_index("c")
    base_w = jnp.minimum(wid * PW, N - PW)

    def chunk(j, carry):
        base = base_w + j * C
        pltpu.sync_copy(idx_hbm.at[pl.ds(base, C)], idx_v)
        pltpu.sync_copy(feat_hbm.at[pl.ds(base, C)], feat_v)
        pltpu.async_copy(table_hbm.at[idx_v], gath_v, sem).wait()

        def row(r, carry2):
            for c in range(D // LANES):
                sl = pl.ds(c * LANES, LANES)
                feat_v[r, sl] = feat_v[r, sl] + gath_v[r, sl]
            return carry2

        lax.fori_loop(0, C, row, 0, unroll=2)
        pltpu.sync_copy(feat_v, out_hbm.at[pl.ds(base, C)])
        return carry

    lax.fori_loop(0, NCHUNK, chunk, 0)


@jax.jit
def _run(type_indices, features, table):
    mesh = plsc.VectorSubcoreMesh(core_axis_name="c", subcore_axis_name="s")
    return pl.kernel(
        _body,
        out_type=jax.ShapeDtypeStruct((N, D), jnp.float32),
        mesh=mesh,
        scratch_types=[
            pltpu.VMEM((C,), jnp.int32),
            pltpu.VMEM((C, D), jnp.float32),
            pltpu.VMEM((C, D), jnp.float32),
            pltpu.SemaphoreType.DMA,
        ],
    )(type_indices, features, table)


def kernel(type_indices, features, type_embedding_weight):
    return _run(type_indices.astype(jnp.int32), features,
                type_embedding_weight)
